# baseline (speedup 1.0000x reference)
import functools

import jax
import jax.numpy as jnp
from jax.experimental import pallas as pl
import jax.experimental.pallas.tpu as pltpu

DIM = 2048
E = 64
EPS = 1e-06


def _router_kernel(x_ref, w_ref, b_ref, vals_ref, loss_ref, hit_s, p0_s,
                   *, n_st, n_b, seq, cap):
    bi = pl.program_id(0)
    st = pl.program_id(1)

    xb = x_ref[0]
    w = w_ref[...]
    logits = jax.lax.dot_general(
        w, xb, (((1,), (1,)), ((), ())),
        preferred_element_type=jnp.float32) + b_ref[...]

    colmax = jnp.max(logits, axis=0, keepdims=True)
    iota = jax.lax.broadcasted_iota(jnp.int32, logits.shape, 0)
    first = jnp.min(jnp.where(logits == colmax, iota, E), axis=0,
                    keepdims=True)
    onehot = (iota == first).astype(jnp.float32)
    hit_part = jnp.max(onehot, axis=1, keepdims=True)

    lane = jax.lax.broadcasted_iota(jnp.int32, hit_s.shape, 1)
    old = hit_s[...]
    upd = jnp.where(st == 0, hit_part, jnp.maximum(old, hit_part))
    hit_s[...] = jnp.where(lane == bi, upd, old)

    @pl.when(st == 0)
    def _():
        cols = logits[:, :E]
        m = jnp.max(cols, axis=0, keepdims=True)
        ex = jnp.exp(cols - m)
        se = jnp.sum(ex, axis=0, keepdims=True)
        p0_s[pl.ds(bi, 1), :] = ex[:1, :] / se

    @pl.when(jnp.logical_and(bi == n_b - 1, st == n_st - 1))
    def _():
        hit = hit_s[...].T
        p0 = p0_s[...]
        masked = p0 * hit
        denom = jnp.sum(masked, axis=0, keepdims=True) + EPS
        vals = masked / denom * cap
        vals_ref[...] = vals
        imp = jnp.sum(vals, axis=0)
        load = jnp.sum((vals > 0).astype(jnp.float32), axis=0)

        n = float(seq * E)
        def cv2(v):
            s1 = jnp.sum(v)
            s2 = jnp.sum(v * v)
            m_ = s1 / n
            var = (s2 - n * m_ * m_) / (n - 1.0)
            return var / (m_ * m_ + 1e-10)

        loss_ref[...] = (cv2(imp) + cv2(load)).reshape(1, 1)


@jax.jit
def kernel(x, W, b):
    B, S, D = x.shape
    ne = W.shape[0]
    cap = float(int(1.0 * B))
    TS = 1024
    n_st = S // TS
    grid = (B, n_st)

    vals, loss = pl.pallas_call(
        functools.partial(_router_kernel, n_st=n_st, n_b=B, seq=S, cap=cap),
        grid=grid,
        in_specs=[
            pl.BlockSpec((1, TS, D), lambda bi, st: (bi, st, 0)),
            pl.BlockSpec((ne, D), lambda bi, st: (0, 0)),
            pl.BlockSpec((ne, 1), lambda bi, st: (0, 0)),
        ],
        out_specs=[
            pl.BlockSpec((B, ne), lambda bi, st: (0, 0)),
            pl.BlockSpec((1, 1), lambda bi, st: (0, 0)),
        ],
        out_shape=[
            jax.ShapeDtypeStruct((B, ne), jnp.float32),
            jax.ShapeDtypeStruct((1, 1), jnp.float32),
        ],
        scratch_shapes=[
            pltpu.VMEM((ne, B), jnp.float32),
            pltpu.VMEM((B, ne), jnp.float32),
        ],
    )(x, W, b.reshape(ne, 1))

    gs = jnp.zeros((B, S, ne), jnp.float32).at[:, :ne, 0].set(vals)
    return gs, loss[0, 0]
